# baseline (speedup 1.0000x reference)
import functools

import jax
import jax.numpy as jnp
from jax import lax
from jax.experimental import pallas as pl
from jax.experimental.pallas import tpu as pltpu
from jax.experimental.pallas import tpu_sc as plsc

NC = 2
NS = 16
LANES = 16
CHUNK = 80
DCHUNK = 80
NBUF = 4


def _make_scatter(N, E, D):
    NW = NC * NS
    epw = E // NW
    cpt = epw // CHUNK
    U = CHUNK
    NP = -(-(N + 1) // U) * U
    nck = NP // U
    assert epw * NW == E and cpt * CHUNK == epw
    assert U % 8 == 0 and D % LANES == 0
    PSZ = 32
    passes = []
    st = 0
    while st < cpt:
        n = min(PSZ, cpt - st)
        passes.append((st, n))
        st += n
    BUF = max(n for _, n in passes)
    assert min(n for _, n in passes) >= NBUF
    assert all(st % 8 == 0 for st, _ in passes)

    mesh = plsc.VectorSubcoreMesh(core_axis_name="c", subcore_axis_name="s")

    @functools.partial(
        pl.kernel,
        out_type=jax.ShapeDtypeStruct((NC, NP, D), jnp.float32),
        mesh=mesh,
        scratch_types=[
            pltpu.VMEM((BUF, CHUNK), jnp.int32),
            pltpu.VMEM((BUF, CHUNK), jnp.int32),
            [pltpu.VMEM((CHUNK, D), jnp.float32) for _ in range(NBUF)],
            pltpu.VMEM_SHARED((NP, D), jnp.float32),
            [pltpu.SemaphoreType.DMA for _ in range(NBUF)],
            [pltpu.SemaphoreType.DMA for _ in range(NBUF)],
        ],
        compiler_params=pltpu.CompilerParams(needs_layout_passes=False),
    )
    def scat(y_hbm, src_hbm, dst_hbm, out_hbm, sidx, didx, rows, acc,
             gsem, ssem):
        c = lax.axis_index("c")
        s = lax.axis_index("s")
        wid = c * NS + s

        zero = jnp.zeros((LANES,), jnp.float32)
        stage = rows[0]

        def zrow(i, _):
            for k in range(D // LANES):
                stage[i, pl.ds(k * LANES, LANES)] = zero
            return 0

        lax.fori_loop(0, U, zrow, 0)
        ntile = (nck - s + NS - 1) // NS

        def zcp(t, _):
            pltpu.sync_copy(stage, acc.at[pl.ds((s + t * NS) * U, U)])
            return 0

        lax.fori_loop(0, ntile, zcp, 0)
        plsc.subcore_barrier()

        def run_pass(start, n):
            if n == BUF:
                pltpu.sync_copy(src_hbm.at[wid, pl.ds(start, n)], sidx)
                pltpu.sync_copy(dst_hbm.at[wid, pl.ds(start, n)], didx)
            else:
                sl = pl.ds(0, n)
                pltpu.sync_copy(src_hbm.at[wid, pl.ds(start, n)], sidx.at[sl])
                pltpu.sync_copy(dst_hbm.at[wid, pl.ds(start, n)], didx.at[sl])
            for b in range(NBUF - 1):
                pltpu.async_copy(y_hbm.at[sidx.at[b]], rows[b], gsem[b])

            def body(j, b):
                bm1 = (b - 1) % NBUF
                pltpu.make_async_copy(
                    y_hbm.at[sidx.at[j]], rows[b], gsem[b]
                ).wait()
                pltpu.async_copy(rows[b], acc.at[didx.at[j]], ssem[b], add=True)
                @pl.when(j > 0)
                def _():
                    pltpu.make_async_copy(
                        rows[bm1], acc.at[didx.at[j]], ssem[bm1]
                    ).wait()

                @pl.when(j + NBUF - 1 < n)
                def _():
                    pltpu.async_copy(
                        y_hbm.at[sidx.at[j + NBUF - 1]], rows[bm1], gsem[bm1]
                    )

            def grp(g, _):
                for b in range(NBUF):
                    body(g * NBUF + b, b)
                return 0

            ngroups = n // NBUF
            lax.fori_loop(0, ngroups, grp, 0)
            for r in range(n % NBUF):
                j = ngroups * NBUF + r
                body(jnp.int32(j), j % NBUF)
            lastb = (n - 1) % NBUF
            pltpu.make_async_copy(
                rows[lastb], acc.at[didx.at[0]], ssem[lastb]
            ).wait()

        for start, n in passes:
            run_pass(start, n)
        plsc.subcore_barrier()

        def wb(t, _):
            sl = pl.ds((s + t * NS) * U, U)
            pltpu.sync_copy(acc.at[sl], out_hbm.at[c, sl])
            return 0

        lax.fori_loop(0, ntile, wb, 0)

    return scat


def _make_deg(N, E):
    NW = NC * NS
    epw = E // NW
    cpt = epw // DCHUNK
    assert cpt * DCHUNK == epw and DCHUNK % LANES == 0
    NR = (-(-N // 128) + LANES - 1) // LANES * LANES
    mesh = plsc.VectorSubcoreMesh(core_axis_name="c", subcore_axis_name="s")

    @functools.partial(
        pl.kernel,
        out_type=jax.ShapeDtypeStruct((NC, NR, 128), jnp.float32),
        mesh=mesh,
        scratch_types=[
            pltpu.VMEM((cpt, DCHUNK), jnp.int32),
            pltpu.VMEM((NR, 128), jnp.float32),
            pltpu.VMEM((NR,), jnp.int32),
            pltpu.VMEM_SHARED((NR, 128), jnp.float32),
        ],
        compiler_params=pltpu.CompilerParams(needs_layout_passes=False),
    )
    def degk(dst_hbm, out_hbm, didx, hist, riota, acc):
        c = lax.axis_index("c")
        s = lax.axis_index("s")
        wid = c * NS + s

        zero = jnp.zeros((LANES,), jnp.float32)
        base = lax.iota(jnp.int32, LANES)

        def zrow(i, _):
            for k in range(128 // LANES):
                hist[i, pl.ds(k * LANES, LANES)] = zero
            return 0

        lax.fori_loop(0, NR, zrow, 0)
        for k in range(NR // LANES):
            riota[pl.ds(k * LANES, LANES)] = base + (k * LANES)

        @pl.when(s == 0)
        def _():
            pltpu.sync_copy(hist, acc)

        ones = jnp.ones((LANES,), jnp.float32)
        pltpu.sync_copy(dst_hbm.at[wid], didx)

        def step(j, _):
            for k in range(DCHUNK // LANES):
                idx = didx[j, pl.ds(k * LANES, LANES)]
                plsc.addupdate_scatter(hist, [idx >> 7, idx & 127], ones)
            return 0

        lax.fori_loop(0, cpt, step, 0)
        plsc.subcore_barrier()
        pltpu.sync_copy(hist, acc.at[riota], add=True)
        plsc.subcore_barrier()

        @pl.when(s == 0)
        def _():
            pltpu.sync_copy(acc, hist)
            pltpu.sync_copy(hist, out_hbm.at[c])

    return degk


def _dinv(d0, d1):
    return lax.rsqrt(d0 + d1 + 1.0)


def _mm_first(x_ref, w_ref, d0_ref, d1_ref, y_ref):
    dinv = _dinv(d0_ref[...], d1_ref[...])
    y_ref[...] = jnp.dot(
        x_ref[...] * dinv, w_ref[...], preferred_element_type=jnp.float32
    )


def _mm_mid(a_ref, y_ref, b_ref, w_ref, d0_ref, d1_ref, o_ref):
    n = y_ref.shape[0]
    dinv = _dinv(d0_ref[...], d1_ref[...])
    acc = a_ref[0, :n, :] + a_ref[1, :n, :]
    h = (acc + y_ref[...]) * dinv + b_ref[...]
    h = jnp.maximum(h, 0.0)
    o_ref[...] = jnp.dot(h * dinv, w_ref[...], preferred_element_type=jnp.float32)


def _mm_last(a_ref, y_ref, b_ref, d0_ref, d1_ref, o_ref):
    n = y_ref.shape[0]
    dinv = _dinv(d0_ref[...], d1_ref[...])
    acc = a_ref[0, :n, :] + a_ref[1, :n, :]
    o_ref[...] = (acc + y_ref[...]) * dinv + b_ref[...]


def kernel(x, edge_index, W1, b1, W2, b2, W3, b3):
    N, D = x.shape
    E = edge_index.shape[1]
    NW = NC * NS
    epw = E // NW
    epw_p = -(-epw // CHUNK) * CHUNK
    s2 = edge_index[0].reshape(NW, epw)
    d2 = edge_index[1].reshape(NW, epw)
    pad = epw_p - epw
    if pad:
        NP = -(-(N + 1) // CHUNK) * CHUNK
        sinks = N + (jnp.arange(pad, dtype=jnp.int32) % (NP - N))
        s2 = jnp.pad(s2, ((0, 0), (0, pad)))
        d2 = jnp.concatenate(
            [d2, jnp.broadcast_to(sinks, (NW, pad))], axis=1)
    src = s2.reshape(NW, epw_p // CHUNK, CHUNK)
    dst = d2.reshape(NW, epw_p // CHUNK, CHUNK)
    dstd = edge_index[1].reshape(NW, E // (NW * DCHUNK), DCHUNK)
    b1r, b2r, b3r = (b.reshape(1, -1) for b in (b1, b2, b3))

    deg = _make_deg(N, E)(dstd)
    d0 = deg[0].reshape(-1)[:N, None]
    d1 = deg[1].reshape(-1)[:N, None]
    scat = _make_scatter(N, NW * epw_p, D)

    shp = jax.ShapeDtypeStruct((N, D), jnp.float32)
    y1 = pl.pallas_call(_mm_first, out_shape=shp)(x, W1, d0, d1)
    a1 = scat(y1, src, dst)
    y2 = pl.pallas_call(_mm_mid, out_shape=shp)(a1, y1, b1r, W2, d0, d1)
    a2 = scat(y2, src, dst)
    y3 = pl.pallas_call(_mm_mid, out_shape=shp)(a2, y2, b2r, W3, d0, d1)
    a3 = scat(y3, src, dst)
    out = pl.pallas_call(_mm_last, out_shape=shp)(a3, y3, b3r, d0, d1)
    return out


# speedup vs baseline: 1.0601x; 1.0601x over previous
import functools

import jax
import jax.numpy as jnp
from jax import lax
from jax.experimental import pallas as pl
from jax.experimental.pallas import tpu as pltpu
from jax.experimental.pallas import tpu_sc as plsc

NC = 2
NS = 16
LANES = 16
CHUNK = 80
DCHUNK = 80
NBUF = 3


def _make_scatter(N, E, D):
    NW = NC * NS
    epw = E // NW
    cpt = epw // CHUNK
    U = CHUNK
    NP = -(-(N + 1) // U) * U
    nck = NP // U
    assert epw * NW == E and cpt * CHUNK == epw
    assert U % 8 == 0 and D % LANES == 0
    PSZ = 64
    passes = []
    st = 0
    while st < cpt:
        n = min(PSZ, cpt - st)
        passes.append((st, n))
        st += n
    BUF = max(n for _, n in passes)
    assert min(n for _, n in passes) >= NBUF
    assert all(st % 8 == 0 for st, _ in passes)

    mesh = plsc.VectorSubcoreMesh(core_axis_name="c", subcore_axis_name="s")

    @functools.partial(
        pl.kernel,
        out_type=jax.ShapeDtypeStruct((NC, NP, D), jnp.float32),
        mesh=mesh,
        scratch_types=[
            pltpu.VMEM((BUF, CHUNK), jnp.int32),
            pltpu.VMEM((BUF, CHUNK), jnp.int32),
            [pltpu.VMEM((CHUNK, D), jnp.float32) for _ in range(NBUF)],
            pltpu.VMEM_SHARED((NP, D), jnp.float32),
            [pltpu.SemaphoreType.DMA for _ in range(NBUF)],
            [pltpu.SemaphoreType.DMA for _ in range(NBUF)],
        ],
        compiler_params=pltpu.CompilerParams(needs_layout_passes=False),
    )
    def scat(y_hbm, src_hbm, dst_hbm, out_hbm, sidx, didx, rows, acc,
             gsem, ssem):
        c = lax.axis_index("c")
        s = lax.axis_index("s")
        wid = c * NS + s

        zero = jnp.zeros((LANES,), jnp.float32)
        stage = rows[0]

        def zrow(i, _):
            for k in range(D // LANES):
                stage[i, pl.ds(k * LANES, LANES)] = zero
            return 0

        lax.fori_loop(0, U, zrow, 0)
        ntile = (nck - s + NS - 1) // NS

        def zcp(t, _):
            pltpu.sync_copy(stage, acc.at[pl.ds((s + t * NS) * U, U)])
            return 0

        lax.fori_loop(0, ntile, zcp, 0)
        plsc.subcore_barrier()

        def run_pass(start, n):
            if n == BUF:
                pltpu.sync_copy(src_hbm.at[wid, pl.ds(start, n)], sidx)
                pltpu.sync_copy(dst_hbm.at[wid, pl.ds(start, n)], didx)
            else:
                sl = pl.ds(0, n)
                pltpu.sync_copy(src_hbm.at[wid, pl.ds(start, n)], sidx.at[sl])
                pltpu.sync_copy(dst_hbm.at[wid, pl.ds(start, n)], didx.at[sl])
            for b in range(NBUF - 1):
                pltpu.async_copy(y_hbm.at[sidx.at[b]], rows[b], gsem[b])

            def body(j, b):
                bm1 = (b - 1) % NBUF
                pltpu.make_async_copy(
                    y_hbm.at[sidx.at[j]], rows[b], gsem[b]
                ).wait()
                pltpu.async_copy(rows[b], acc.at[didx.at[j]], ssem[b], add=True)
                @pl.when(j > 0)
                def _():
                    pltpu.make_async_copy(
                        rows[bm1], acc.at[didx.at[j]], ssem[bm1]
                    ).wait()

                @pl.when(j + NBUF - 1 < n)
                def _():
                    pltpu.async_copy(
                        y_hbm.at[sidx.at[j + NBUF - 1]], rows[bm1], gsem[bm1]
                    )

            def grp(g, _):
                for b in range(NBUF):
                    body(g * NBUF + b, b)
                return 0

            ngroups = n // NBUF
            lax.fori_loop(0, ngroups, grp, 0)
            for r in range(n % NBUF):
                j = ngroups * NBUF + r
                body(jnp.int32(j), j % NBUF)
            lastb = (n - 1) % NBUF
            pltpu.make_async_copy(
                rows[lastb], acc.at[didx.at[0]], ssem[lastb]
            ).wait()

        for start, n in passes:
            run_pass(start, n)
        plsc.subcore_barrier()

        def wb(t, _):
            sl = pl.ds((s + t * NS) * U, U)
            pltpu.sync_copy(acc.at[sl], out_hbm.at[c, sl])
            return 0

        lax.fori_loop(0, ntile, wb, 0)

    return scat


def _make_deg(N, E):
    NW = NC * NS
    epw = E // NW
    cpt = epw // DCHUNK
    assert cpt * DCHUNK == epw and DCHUNK % LANES == 0
    NR = (-(-N // 128) + LANES - 1) // LANES * LANES
    mesh = plsc.VectorSubcoreMesh(core_axis_name="c", subcore_axis_name="s")

    @functools.partial(
        pl.kernel,
        out_type=jax.ShapeDtypeStruct((NC, NR, 128), jnp.float32),
        mesh=mesh,
        scratch_types=[
            pltpu.VMEM((cpt, DCHUNK), jnp.int32),
            pltpu.VMEM((NR, 128), jnp.float32),
            pltpu.VMEM((NR,), jnp.int32),
            pltpu.VMEM_SHARED((NR, 128), jnp.float32),
        ],
        compiler_params=pltpu.CompilerParams(needs_layout_passes=False),
    )
    def degk(dst_hbm, out_hbm, didx, hist, riota, acc):
        c = lax.axis_index("c")
        s = lax.axis_index("s")
        wid = c * NS + s

        zero = jnp.zeros((LANES,), jnp.float32)
        base = lax.iota(jnp.int32, LANES)

        def zrow(i, _):
            for k in range(128 // LANES):
                hist[i, pl.ds(k * LANES, LANES)] = zero
            return 0

        lax.fori_loop(0, NR, zrow, 0)
        for k in range(NR // LANES):
            riota[pl.ds(k * LANES, LANES)] = base + (k * LANES)

        @pl.when(s == 0)
        def _():
            pltpu.sync_copy(hist, acc)

        ones = jnp.ones((LANES,), jnp.float32)
        pltpu.sync_copy(dst_hbm.at[wid], didx)

        def step(j, _):
            for k in range(DCHUNK // LANES):
                idx = didx[j, pl.ds(k * LANES, LANES)]
                plsc.addupdate_scatter(hist, [idx >> 7, idx & 127], ones)
            return 0

        lax.fori_loop(0, cpt, step, 0)
        plsc.subcore_barrier()
        pltpu.sync_copy(hist, acc.at[riota], add=True)
        plsc.subcore_barrier()

        @pl.when(s == 0)
        def _():
            pltpu.sync_copy(acc, hist)
            pltpu.sync_copy(hist, out_hbm.at[c])

    return degk


def _dinv(d0, d1):
    return lax.rsqrt(d0 + d1 + 1.0)


def _mm_first(x_ref, w_ref, d0_ref, d1_ref, y_ref):
    dinv = _dinv(d0_ref[...], d1_ref[...])
    y_ref[...] = jnp.dot(
        x_ref[...] * dinv, w_ref[...], preferred_element_type=jnp.float32
    )


def _mm_mid(a_ref, y_ref, b_ref, w_ref, d0_ref, d1_ref, o_ref):
    n = y_ref.shape[0]
    dinv = _dinv(d0_ref[...], d1_ref[...])
    acc = a_ref[0, :n, :] + a_ref[1, :n, :]
    h = (acc + y_ref[...]) * dinv + b_ref[...]
    h = jnp.maximum(h, 0.0)
    o_ref[...] = jnp.dot(h * dinv, w_ref[...], preferred_element_type=jnp.float32)


def _mm_last(a_ref, y_ref, b_ref, d0_ref, d1_ref, o_ref):
    n = y_ref.shape[0]
    dinv = _dinv(d0_ref[...], d1_ref[...])
    acc = a_ref[0, :n, :] + a_ref[1, :n, :]
    o_ref[...] = (acc + y_ref[...]) * dinv + b_ref[...]


def kernel(x, edge_index, W1, b1, W2, b2, W3, b3):
    N, D = x.shape
    E = edge_index.shape[1]
    NW = NC * NS
    epw = E // NW
    epw_p = -(-epw // CHUNK) * CHUNK
    s2 = edge_index[0].reshape(NW, epw)
    d2 = edge_index[1].reshape(NW, epw)
    pad = epw_p - epw
    if pad:
        NP = -(-(N + 1) // CHUNK) * CHUNK
        sinks = N + (jnp.arange(pad, dtype=jnp.int32) % (NP - N))
        s2 = jnp.pad(s2, ((0, 0), (0, pad)))
        d2 = jnp.concatenate(
            [d2, jnp.broadcast_to(sinks, (NW, pad))], axis=1)
    src = s2.reshape(NW, epw_p // CHUNK, CHUNK)
    dst = d2.reshape(NW, epw_p // CHUNK, CHUNK)
    dstd = edge_index[1].reshape(NW, E // (NW * DCHUNK), DCHUNK)
    b1r, b2r, b3r = (b.reshape(1, -1) for b in (b1, b2, b3))

    deg = _make_deg(N, E)(dstd)
    d0 = deg[0].reshape(-1)[:N, None]
    d1 = deg[1].reshape(-1)[:N, None]
    scat = _make_scatter(N, NW * epw_p, D)

    shp = jax.ShapeDtypeStruct((N, D), jnp.float32)
    y1 = pl.pallas_call(_mm_first, out_shape=shp)(x, W1, d0, d1)
    a1 = scat(y1, src, dst)
    y2 = pl.pallas_call(_mm_mid, out_shape=shp)(a1, y1, b1r, W2, d0, d1)
    a2 = scat(y2, src, dst)
    y3 = pl.pallas_call(_mm_mid, out_shape=shp)(a2, y2, b2r, W3, d0, d1)
    a3 = scat(y3, src, dst)
    out = pl.pallas_call(_mm_last, out_shape=shp)(a3, y3, b3r, d0, d1)
    return out
